# baseline (speedup 1.0000x reference)
import functools

import jax
import jax.numpy as jnp
from jax import lax
from jax.experimental import pallas as pl
from jax.experimental.pallas import tpu as pltpu
from jax.experimental.pallas import tpu_sc as plsc

_B = 16384
_F = 32
_NU = 1000000
_CHU = 1024
_NFULL = 976
_TAILC = _NFULL
_SROW = _B + 1

_mesh = plsc.VectorSubcoreMesh(core_axis_name="c", subcore_axis_name="s")
_CP = pltpu.CompilerParams(needs_layout_passes=False, use_tc_tiling_on_sc=True)


@functools.partial(
    pl.kernel,
    mesh=_mesh,
    out_type=(jax.ShapeDtypeStruct((_SROW, 128), jnp.float32),
              jax.ShapeDtypeStruct((_SROW, 128), jnp.float32)),
    scratch_types=[
        pltpu.VMEM((_B,), jnp.int32),
        pltpu.VMEM((_B,), jnp.int32),
        pltpu.VMEM((1024,), jnp.int32),
        pltpu.VMEM((1024,), jnp.int32),
        pltpu.VMEM((1024,), jnp.int32),
        pltpu.VMEM((1024,), jnp.int32),
        pltpu.VMEM((2, 4, 8, _CHU), jnp.float32),
        pltpu.VMEM((4, 8, 512), jnp.float32),
        pltpu.VMEM((4, 8, 64), jnp.float32),
        pltpu.VMEM((128,), jnp.int32),
        pltpu.VMEM((128,), jnp.int32),
        pltpu.VMEM((16, 128), jnp.float32),
        pltpu.VMEM((16,), jnp.int32),
        pltpu.SemaphoreType.DMA,
        pltpu.SemaphoreType.DMA,
        pltpu.SemaphoreType.DMA,
    ],
    compiler_params=_CP,
)
def _mf_stage(user_hbm, movie_hbm, uft_hbm, mft_hbm, ustage, mstage,
              ul_v, ml_v, myu_v, myb_v, mmu_v, mmb_v,
              chunk_v, t5_v, t6_v, wlu_v, wlb_v, tmp_v, cvec_v,
              csem, csem2, ssem):
    wid = lax.axis_index("s") * 2 + lax.axis_index("c")
    iota16 = lax.iota(jnp.int32, 16)
    widv = jnp.full((16,), 0, jnp.int32) + wid

    pltpu.sync_copy(user_hbm, ul_v)
    pltpu.sync_copy(movie_hbm, ml_v)

    def prefill(i, carry):
        myb_v[pl.ds(i * 16, 16)] = jnp.full((16,), _B, jnp.int32)
        mmb_v[pl.ds(i * 16, 16)] = jnp.full((16,), _B, jnp.int32)
        myu_v[pl.ds(i * 16, 16)] = jnp.full((16,), 1 << 30, jnp.int32)
        mmu_v[pl.ds(i * 16, 16)] = jnp.full((16,), 1 << 30, jnp.int32)
        return carry
    lax.fori_loop(0, 64, prefill, 0)

    def bucket(i, cnts):
        ucnt, mcnt = cnts
        u = ul_v[pl.ds(i * 16, 16)]
        m = ml_v[pl.ds(i * 16, 16)]
        b = iota16 + i * 16
        umask = ((u >> 10) & 31) == widv
        mmask = ((m >> 10) & 31) == widv
        plsc.store_compressed(myu_v.at[pl.ds(ucnt, 16)], u, mask=umask)
        plsc.store_compressed(myb_v.at[pl.ds(ucnt, 16)], b, mask=umask)
        plsc.store_compressed(mmu_v.at[pl.ds(mcnt, 16)], m, mask=mmask)
        plsc.store_compressed(mmb_v.at[pl.ds(mcnt, 16)], b, mask=mmask)
        return (ucnt + jnp.sum(umask.astype(jnp.int32)),
                mcnt + jnp.sum(mmask.astype(jnp.int32)))
    lax.fori_loop(0, _B // 16, bucket,
                  (jnp.zeros((), jnp.int32), jnp.zeros((), jnp.int32)))

    def stream_table(tab_hbm, stage_hbm, lu_v, lb_v):
        cvec_v[pl.ds(0, 16)] = widv

        def extract(cvec, gather_vals):
            for j in range(8):
                wlu_v[pl.ds(j * 16, 16)] = cvec * _CHU
                wlb_v[pl.ds(j * 16, 16)] = jnp.full((16,), _B, jnp.int32)

            def build_wl(j, cnt2):
                uu = lu_v[pl.ds(j * 16, 16)]
                bb = lb_v[pl.ds(j * 16, 16)]
                mask = (uu >> 10) == cvec
                plsc.store_compressed(wlu_v.at[pl.ds(cnt2, 16)], uu, mask=mask)
                plsc.store_compressed(wlb_v.at[pl.ds(cnt2, 16)], bb, mask=mask)
                return cnt2 + jnp.sum(mask.astype(jnp.int32))
            cnt2 = lax.fori_loop(0, 64, build_wl, jnp.zeros((), jnp.int32))

            for g in range(4):
                @pl.when(cnt2 > g * 16)
                def _():
                    wlu = wlu_v[pl.ds(g * 16, 16)]
                    wlb = wlb_v[pl.ds(g * 16, 16)]
                    u_loc = jnp.clip(wlu - cvec * _CHU, 0, _CHU - 1)
                    for f in range(_F):
                        plsc.store_scatter(
                            tmp_v,
                            [iota16, jnp.full((16,), f, jnp.int32)],
                            gather_vals(f, u_loc))
                    pltpu.async_copy(tmp_v, stage_hbm.at[wlb],
                                     ssem).wait()

        def gather_chunk(buf):
            def gv(f, u_loc):
                return plsc.load_gather(
                    chunk_v,
                    [jnp.full((16,), buf, jnp.int32),
                     jnp.full((16,), f // 8, jnp.int32),
                     jnp.full((16,), f % 8, jnp.int32), u_loc])
            return gv

        def gather_tail(f, u_loc):
            sel = u_loc < 512
            v5 = plsc.load_gather(
                t5_v, [jnp.full((16,), f // 8, jnp.int32),
                       jnp.full((16,), f % 8, jnp.int32),
                       jnp.clip(u_loc, 0, 511)])
            v6 = plsc.load_gather(
                t6_v, [jnp.full((16,), f // 8, jnp.int32),
                       jnp.full((16,), f % 8, jnp.int32),
                       jnp.clip(u_loc - 512, 0, 63)])
            return jnp.where(sel, v5, v6)

        def start_chunk(c, buf, sem):
            off = pl.multiple_of(c * _CHU, 128)
            for g in range(4):
                pltpu.async_copy(
                    tab_hbm.at[pl.ds(8 * g, 8), pl.ds(off, _CHU)],
                    chunk_v.at[buf, g], sem)

        def wait_chunk(c, buf, sem):
            off = pl.multiple_of(c * _CHU, 128)
            for g in range(4):
                pltpu.make_async_copy(
                    tab_hbm.at[pl.ds(8 * g, 8), pl.ds(off, _CHU)],
                    chunk_v.at[buf, g], sem).wait()

        def start_tail(sem):
            for g in range(4):
                pltpu.async_copy(
                    tab_hbm.at[pl.ds(8 * g, 8), pl.ds(999424, 512)],
                    t5_v.at[g], sem)
                pltpu.async_copy(
                    tab_hbm.at[pl.ds(8 * g, 8), pl.ds(999936, 64)],
                    t6_v.at[g], sem)

        def wait_tail(sem):
            for g in range(4):
                pltpu.make_async_copy(
                    tab_hbm.at[pl.ds(8 * g, 8), pl.ds(999424, 512)],
                    t5_v.at[g], sem).wait()
                pltpu.make_async_copy(
                    tab_hbm.at[pl.ds(8 * g, 8), pl.ds(999936, 64)],
                    t6_v.at[g], sem).wait()

        def pair_body(ci2, carry):
            cvec_a = cvec_v[pl.ds(0, 16)]
            cvec_b = cvec_a + 32
            ca = (ci2 * 2) * 32 + wid
            cb = (ci2 * 2 + 1) * 32 + wid

            pl.when(ca < _NFULL)(lambda: start_chunk(ca, 0, csem))
            pl.when(ca == _TAILC)(lambda: start_tail(csem))
            pl.when(cb < _NFULL)(lambda: start_chunk(cb, 1, csem2))
            pl.when(cb == _TAILC)(lambda: start_tail(csem2))

            @pl.when(ca < _NFULL)
            def _():
                wait_chunk(ca, 0, csem)
                extract(cvec_a, gather_chunk(0))
            @pl.when(ca == _TAILC)
            def _():
                wait_tail(csem)
                extract(cvec_a, gather_tail)
            @pl.when(cb < _NFULL)
            def _():
                wait_chunk(cb, 1, csem2)
                extract(cvec_b, gather_chunk(1))
            @pl.when(cb == _TAILC)
            def _():
                wait_tail(csem2)
                extract(cvec_b, gather_tail)

            cvec_v[pl.ds(0, 16)] = cvec_a + 64
            return carry

        lax.fori_loop(0, 16, pair_body, 0)

    stream_table(uft_hbm, ustage, myu_v, myb_v)
    stream_table(mft_hbm, mstage, mmu_v, mmb_v)


@functools.partial(
    pl.kernel,
    mesh=_mesh,
    out_type=jax.ShapeDtypeStruct((_B,), jnp.float32),
    scratch_types=[
        pltpu.VMEM((256, 128), jnp.float32),
        pltpu.VMEM((256, 128), jnp.float32),
        pltpu.VMEM((512,), jnp.float32),
        pltpu.SemaphoreType.DMA,
    ],
    compiler_params=_CP,
)
def _mf_reduce(ustage, mstage, out_hbm, ub_v, mb_v, out_v, sem):
    wid = lax.axis_index("s") * 2 + lax.axis_index("c")
    base = wid * 512
    iota16 = lax.iota(jnp.int32, 16)

    for p in range(2):
        row0 = base + p * 256
        pltpu.sync_copy(ustage.at[pl.ds(row0, 256), pl.ds(0, 128)], ub_v)
        pltpu.sync_copy(mstage.at[pl.ds(row0, 256), pl.ds(0, 128)], mb_v)

        def g_body(g, carry):
            res = jnp.zeros((16,), jnp.float32)
            for j in range(16):
                r = g * 16 + j
                prod = (ub_v[r, pl.ds(0, 16)] * mb_v[r, pl.ds(0, 16)]
                        + ub_v[r, pl.ds(16, 16)] * mb_v[r, pl.ds(16, 16)])
                res = jnp.where(iota16 == j, jnp.sum(prod), res)
            out_v[pl.ds(p * 256 + g * 16, 16)] = res
            return carry
        lax.fori_loop(0, 16, g_body, 0)

    pltpu.sync_copy(out_v, out_hbm.at[pl.ds(base, 512)])


def kernel(user, movie, user_factors, movie_factors):
    su, sm = _mf_stage(user.astype(jnp.int32), movie.astype(jnp.int32),
                       user_factors.T, movie_factors.T)
    return _mf_reduce(su, sm)
